# Initial kernel scaffold; baseline (speedup 1.0000x reference)
import functools
import math

import jax
import jax.numpy as jnp
from jax import lax
from jax.experimental import pallas as pl
from jax.experimental.pallas import tpu as pltpu
from jax.experimental.pallas import tpu_sc as plsc

_N_LEVELS = 16
_F = 2
_T = 2 ** 19
_BASE_RES = 16
_MAX_RES = 512
_B = 524288
_SCALE = math.exp(math.log(_MAX_RES / _BASE_RES) / (_N_LEVELS - 1))
_RES = []
_OFF = []
_tot = 0
for _i in range(_N_LEVELS):
    _OFF.append(_tot)
    _r = math.floor(_BASE_RES * _SCALE ** _i)
    _RES.append(_r)
    _tot += min(_T, (_r + 1) ** 3)
_OFF.append(_tot)
_N_TOTAL = _tot
_SIZES = [_OFF[i + 1] - _OFF[i] for i in range(_N_LEVELS)]
_P1 = 2654435761
_P2 = 805459861
_OUT_D = 3 + 2 * _N_LEVELS

_NW = 32
_PW = _B // _NW
_C = 1024
_NCH = _PW // _C
_G = _C // 16


def _body(xyz_t, emb, out, xb, yb, zb, fxb, fyb, fzb, idxr, vals, ob, sem):
    del sem
    wid = lax.axis_index("s") * 2 + lax.axis_index("c")
    iota = lax.iota(jnp.int32, 16)

    def chunk_body(ch, carry):
        base = wid * _PW + ch * _C
        pltpu.sync_copy(xyz_t.at[0, pl.ds(base, _C)], xb)
        pltpu.sync_copy(xyz_t.at[1, pl.ds(base, _C)], yb)
        pltpu.sync_copy(xyz_t.at[2, pl.ds(base, _C)], zb)

        def pt_body(g, c2):
            p = g * 16 + iota
            for col, buf in ((0, xb), (1, yb), (2, zb)):
                v = buf[pl.ds(g * 16, 16)]
                plsc.store_scatter(ob, [p, jnp.full((16,), col, jnp.int32)], v)
            return c2

        lax.fori_loop(0, _G, pt_body, 0)

        for l in range(_N_LEVELS):
            res = _RES[l]
            off = _OFF[l]
            size = _SIZES[l]

            def p1(g, c2, res=res, off=off, size=size):
                s16 = pl.ds(g * 16, 16)

                def prep(v):
                    vn = jnp.minimum(jnp.maximum(v, jnp.float32(0.0)),
                                     jnp.float32(1.0))
                    pos = vn * jnp.float32(res)
                    gi = pos.astype(jnp.int32)
                    gi = jnp.minimum(gi, jnp.int32(res - 1))
                    fr = pos - gi.astype(jnp.float32)
                    return gi.astype(jnp.uint32), fr

                gx, fx = prep(xb[s16])
                gy, fy = prep(yb[s16])
                gz, fz = prep(zb[s16])
                fxb[s16] = fx
                fyb[s16] = fy
                fzb[s16] = fz
                hx = (gx, gx + jnp.uint32(1))
                hy0 = gy * jnp.uint32(_P1)
                hy = (hy0, hy0 + jnp.uint32(_P1))
                hz0 = gz * jnp.uint32(_P2)
                hz = (hz0, hz0 + jnp.uint32(_P2))
                hyz = (hy[0] ^ hz[0], hy[0] ^ hz[1],
                       hy[1] ^ hz[0], hy[1] ^ hz[1])
                for k in range(8):
                    cx = (k >> 2) & 1
                    h = hx[cx] ^ hyz[k & 3]
                    if size == _T:
                        r = h & jnp.uint32(_T - 1)
                    else:
                        hf = h.astype(jnp.float32)
                        q = (hf * jnp.float32(1.0 / size)).astype(jnp.int32)
                        qu = lax.bitcast_convert_type(q, jnp.uint32)
                        r = h - qu * jnp.uint32(size)
                        ri = lax.bitcast_convert_type(r, jnp.int32)
                        r = jnp.where(ri < 0, r + jnp.uint32(size), r)
                        r = jnp.where(r >= jnp.uint32(size),
                                      r - jnp.uint32(size), r)
                    hidx = lax.bitcast_convert_type(r, jnp.int32) + off
                    idxr[k * _G + g] = hidx
                return c2

            lax.fori_loop(0, _G, p1, 0)
            pltpu.sync_copy(emb.at[idxr], vals)

            def p2(g, c2, l=l):
                s16 = pl.ds(g * 16, 16)
                fx = fxb[s16]
                fy = fyb[s16]
                fz = fzb[s16]
                one = jnp.float32(1.0)
                wx = (one - fx, fx)
                wy = (one - fy, fy)
                wz = (one - fz, fz)
                c0 = jnp.full((16,), 0, jnp.int32)
                c1 = jnp.full((16,), 1, jnp.int32)
                acc0 = jnp.zeros((16,), jnp.float32)
                acc1 = jnp.zeros((16,), jnp.float32)
                for k in range(8):
                    cx = (k >> 2) & 1
                    cy = (k >> 1) & 1
                    cz = k & 1
                    w = (wx[cx] * wy[cy]) * wz[cz]
                    row = jnp.full((16,), k * _G, jnp.int32) + g
                    v0 = plsc.load_gather(vals, [row, iota, c0])
                    v1 = plsc.load_gather(vals, [row, iota, c1])
                    acc0 = acc0 + w * v0
                    acc1 = acc1 + w * v1
                p = g * 16 + iota
                plsc.store_scatter(
                    ob, [p, jnp.full((16,), 3 + 2 * l, jnp.int32)], acc0)
                plsc.store_scatter(
                    ob, [p, jnp.full((16,), 4 + 2 * l, jnp.int32)], acc1)
                return c2

            lax.fori_loop(0, _G, p2, 0)

        pltpu.sync_copy(ob, out.at[pl.ds(base, _C)])
        return carry

    lax.fori_loop(0, _NCH, chunk_body, 0)


_sc_call = pl.kernel(
    _body,
    out_type=jax.ShapeDtypeStruct((_B, _OUT_D), jnp.float32),
    mesh=plsc.VectorSubcoreMesh(core_axis_name="c", subcore_axis_name="s"),
    scratch_types=[
        pltpu.VMEM((_C,), jnp.float32),
        pltpu.VMEM((_C,), jnp.float32),
        pltpu.VMEM((_C,), jnp.float32),
        pltpu.VMEM((_C,), jnp.float32),
        pltpu.VMEM((_C,), jnp.float32),
        pltpu.VMEM((_C,), jnp.float32),
        pltpu.VMEM((8 * _G, 16), jnp.int32),
        pltpu.VMEM((8 * _G, 16, _F), jnp.float32),
        pltpu.VMEM((_C, _OUT_D), jnp.float32),
        pltpu.SemaphoreType.DMA,
    ],
)


@jax.jit
def kernel(xyz, embeddings):
    xyz_t = xyz.T
    return _sc_call(xyz_t, embeddings)


# speedup vs baseline: 2.7727x; 2.7727x over previous
import math

import jax
import jax.numpy as jnp
from jax import lax
from jax.experimental import pallas as pl
from jax.experimental.pallas import tpu as pltpu
from jax.experimental.pallas import tpu_sc as plsc

_N_LEVELS = 16
_F = 2
_T = 2 ** 19
_BASE_RES = 16
_MAX_RES = 512
_B = 524288
_SCALE = math.exp(math.log(_MAX_RES / _BASE_RES) / (_N_LEVELS - 1))
_RES = []
_OFF = []
_tot = 0
for _i in range(_N_LEVELS):
    _OFF.append(_tot)
    _r = math.floor(_BASE_RES * _SCALE ** _i)
    _RES.append(_r)
    _tot += min(_T, (_r + 1) ** 3)
_OFF.append(_tot)
_N_TOTAL = _tot
_SIZES = [_OFF[i + 1] - _OFF[i] for i in range(_N_LEVELS)]
_P1 = 2654435761
_P2 = 805459861
_OUT_D = 3 + 2 * _N_LEVELS

_NW = 32
_PW = _B // _NW
_C = 1024
_NCH = _PW // _C
_G = _C // 16


def _body(xh, yh, zh, emb, out, xb, yb, zb, fxb, fyb, fzb, idxr, vals, ob,
          sem):
    del sem
    wid = lax.axis_index("s") * 2 + lax.axis_index("c")
    iota = lax.iota(jnp.int32, 16)
    iota35 = iota * _OUT_D

    def chunk_body(ch, carry):
        base = wid * _PW + ch * _C
        pltpu.sync_copy(xh.at[pl.ds(base, _C)], xb)
        pltpu.sync_copy(yh.at[pl.ds(base, _C)], yb)
        pltpu.sync_copy(zh.at[pl.ds(base, _C)], zb)

        def pt_body(g, c2):
            pb = iota35 + g * (16 * _OUT_D)
            for col, buf in ((0, xb), (1, yb), (2, zb)):
                v = buf[pl.ds(g * 16, 16)]
                plsc.store_scatter(ob, [pb + col], v)
            return c2

        lax.fori_loop(0, _G, pt_body, 0)

        for l in range(_N_LEVELS):
            res = _RES[l]
            off = _OFF[l]
            size = _SIZES[l]

            def p1(g, c2, res=res, off=off, size=size):
                s16 = pl.ds(g * 16, 16)

                def prep(v):
                    vn = jnp.minimum(jnp.maximum(v, jnp.float32(0.0)),
                                     jnp.float32(1.0))
                    pos = vn * jnp.float32(res)
                    gi = pos.astype(jnp.int32)
                    gi = jnp.minimum(gi, jnp.int32(res - 1))
                    fr = pos - gi.astype(jnp.float32)
                    return gi.astype(jnp.uint32), fr

                gx, fx = prep(xb[s16])
                gy, fy = prep(yb[s16])
                gz, fz = prep(zb[s16])
                fxb[s16] = fx
                fyb[s16] = fy
                fzb[s16] = fz
                hx = (gx, gx + jnp.uint32(1))
                hy0 = gy * jnp.uint32(_P1)
                hy = (hy0, hy0 + jnp.uint32(_P1))
                hz0 = gz * jnp.uint32(_P2)
                hz = (hz0, hz0 + jnp.uint32(_P2))
                hyz = (hy[0] ^ hz[0], hy[0] ^ hz[1],
                       hy[1] ^ hz[0], hy[1] ^ hz[1])
                for k in range(8):
                    cx = (k >> 2) & 1
                    h = hx[cx] ^ hyz[k & 3]
                    if size == _T:
                        r = h & jnp.uint32(_T - 1)
                    else:
                        hf = h.astype(jnp.float32)
                        q = (hf * jnp.float32(1.0 / size)).astype(jnp.int32)
                        qu = lax.bitcast_convert_type(q, jnp.uint32)
                        r = h - qu * jnp.uint32(size)
                        ri = lax.bitcast_convert_type(r, jnp.int32)
                        r = jnp.where(ri < 0, r + jnp.uint32(size), r)
                        r = jnp.where(r >= jnp.uint32(size),
                                      r - jnp.uint32(size), r)
                    hidx = lax.bitcast_convert_type(r, jnp.int32) + off
                    idxr[pl.ds(k * _C + g * 16, 16)] = hidx
                return c2

            lax.fori_loop(0, _G, p1, 0)
            pltpu.sync_copy(emb.at[idxr], vals)

            def p2(g, c2, l=l):
                s16 = pl.ds(g * 16, 16)
                fx = fxb[s16]
                fy = fyb[s16]
                fz = fzb[s16]
                one = jnp.float32(1.0)
                wx = (one - fx, fx)
                wy = (one - fy, fy)
                wz = (one - fz, fz)
                c0 = jnp.full((16,), 0, jnp.int32)
                c1 = jnp.full((16,), 1, jnp.int32)
                acc0 = jnp.zeros((16,), jnp.float32)
                acc1 = jnp.zeros((16,), jnp.float32)
                for k in range(8):
                    cx = (k >> 2) & 1
                    cy = (k >> 1) & 1
                    cz = k & 1
                    w = (wx[cx] * wy[cy]) * wz[cz]
                    row = g * 16 + iota + (k * _C)
                    v0 = plsc.load_gather(vals, [row, c0])
                    v1 = plsc.load_gather(vals, [row, c1])
                    acc0 = acc0 + w * v0
                    acc1 = acc1 + w * v1
                pb = iota35 + g * (16 * _OUT_D)
                plsc.store_scatter(ob, [pb + (3 + 2 * l)], acc0)
                plsc.store_scatter(ob, [pb + (4 + 2 * l)], acc1)
                return c2

            lax.fori_loop(0, _G, p2, 0)

        pltpu.sync_copy(ob, out.at[pl.ds(base * _OUT_D, _C * _OUT_D)])
        return carry

    lax.fori_loop(0, _NCH, chunk_body, 0)


_sc_call = pl.kernel(
    _body,
    out_type=jax.ShapeDtypeStruct((_B * _OUT_D,), jnp.float32),
    mesh=plsc.VectorSubcoreMesh(core_axis_name="c", subcore_axis_name="s"),
    compiler_params=pltpu.CompilerParams(
        needs_layout_passes=False, use_tc_tiling_on_sc=False),
    scratch_types=[
        pltpu.VMEM((_C,), jnp.float32),
        pltpu.VMEM((_C,), jnp.float32),
        pltpu.VMEM((_C,), jnp.float32),
        pltpu.VMEM((_C,), jnp.float32),
        pltpu.VMEM((_C,), jnp.float32),
        pltpu.VMEM((_C,), jnp.float32),
        pltpu.VMEM((8 * _C,), jnp.int32),
        pltpu.VMEM((8 * _C, _F), jnp.float32),
        pltpu.VMEM((_C * _OUT_D,), jnp.float32),
        pltpu.SemaphoreType.DMA,
    ],
)


@jax.jit
def kernel(xyz, embeddings):
    x = xyz[:, 0]
    y = xyz[:, 1]
    z = xyz[:, 2]
    flat = _sc_call(x, y, z, embeddings)
    return flat.reshape(_B, _OUT_D)


# speedup vs baseline: 2.8651x; 1.0333x over previous
import math

import jax
import jax.numpy as jnp
from jax import lax
from jax.experimental import pallas as pl
from jax.experimental.pallas import tpu as pltpu
from jax.experimental.pallas import tpu_sc as plsc

_N_LEVELS = 16
_F = 2
_T = 2 ** 19
_BASE_RES = 16
_MAX_RES = 512
_B = 524288
_SCALE = math.exp(math.log(_MAX_RES / _BASE_RES) / (_N_LEVELS - 1))
_RES = []
_OFF = []
_tot = 0
for _i in range(_N_LEVELS):
    _OFF.append(_tot)
    _r = math.floor(_BASE_RES * _SCALE ** _i)
    _RES.append(_r)
    _tot += min(_T, (_r + 1) ** 3)
_OFF.append(_tot)
_N_TOTAL = _tot
_SIZES = [_OFF[i + 1] - _OFF[i] for i in range(_N_LEVELS)]
_P1 = 2654435761
_P2 = 805459861
_OUT_D = 3 + 2 * _N_LEVELS

_NW = 32
_PW = _B // _NW
_C = 1024
_NCH = _PW // _C
_G = _C // 16

_N_RES_LEVELS = 1
_TAB_WORDS = _OFF[_N_RES_LEVELS] * _F


def _hash_corners(gx, gy, gz):
    hx = (gx, gx + jnp.uint32(1))
    hy0 = gy * jnp.uint32(_P1)
    hy = (hy0, hy0 + jnp.uint32(_P1))
    hz0 = gz * jnp.uint32(_P2)
    hz = (hz0, hz0 + jnp.uint32(_P2))
    hyz = (hy[0] ^ hz[0], hy[0] ^ hz[1], hy[1] ^ hz[0], hy[1] ^ hz[1])
    return tuple(hx[(k >> 2) & 1] ^ hyz[k & 3] for k in range(8))


def _mod_level(h, size, off):
    if size == _T:
        r = h & jnp.uint32(_T - 1)
    else:
        hf = h.astype(jnp.float32)
        q = (hf * jnp.float32(1.0 / size)).astype(jnp.int32)
        qu = lax.bitcast_convert_type(q, jnp.uint32)
        r = h - qu * jnp.uint32(size)
        ri = lax.bitcast_convert_type(r, jnp.int32)
        r = jnp.where(ri < 0, r + jnp.uint32(size), r)
        r = jnp.where(r >= jnp.uint32(size), r - jnp.uint32(size), r)
    hidx = lax.bitcast_convert_type(r, jnp.int32) + off
    return hidx + hidx


def _prep(v, res):
    vn = jnp.minimum(jnp.maximum(v, jnp.float32(0.0)), jnp.float32(1.0))
    pos = vn * jnp.float32(res)
    gi = pos.astype(jnp.int32)
    gi = jnp.minimum(gi, jnp.int32(res - 1))
    fr = pos - gi.astype(jnp.float32)
    return gi.astype(jnp.uint32), fr


def _corner_w(fx, fy, fz):
    one = jnp.float32(1.0)
    wx = (one - fx, fx)
    wy = (one - fy, fy)
    wz = (one - fz, fz)
    return tuple((wx[(k >> 2) & 1] * wy[(k >> 1) & 1]) * wz[k & 1]
                 for k in range(8))


def _body(xh, yh, zh, emb, out, xb, yb, zb, f0x, f0y, f0z, f1x, f1y, f1z,
          idx0, idx1, vals0, vals1, tab, ob, sem0, sem1):
    wid = lax.axis_index("s") * 2 + lax.axis_index("c")
    iota = lax.iota(jnp.int32, 16)
    iota35 = iota * _OUT_D
    one_i = jnp.full((16,), 1, jnp.int32)
    fbufs = ((f0x, f0y, f0z), (f1x, f1y, f1z))
    ibufs = (idx0, idx1)
    vbufs = (vals0, vals1)
    sems = (sem0, sem1)

    pltpu.sync_copy(emb.at[pl.ds(0, _TAB_WORDS)], tab)

    def chunk_body(ch, carry):
        base = wid * _PW + ch * _C
        pltpu.sync_copy(xh.at[pl.ds(base, _C)], xb)
        pltpu.sync_copy(yh.at[pl.ds(base, _C)], yb)
        pltpu.sync_copy(zh.at[pl.ds(base, _C)], zb)

        def fused(g, c2):
            s16 = pl.ds(g * 16, 16)
            x = xb[s16]
            y = yb[s16]
            z = zb[s16]
            pb = iota35 + g * (16 * _OUT_D)
            plsc.store_scatter(ob, [pb + 0], x)
            plsc.store_scatter(ob, [pb + 1], y)
            plsc.store_scatter(ob, [pb + 2], z)
            for l in range(_N_RES_LEVELS):
                gx, fx = _prep(x, _RES[l])
                gy, fy = _prep(y, _RES[l])
                gz, fz = _prep(z, _RES[l])
                hs = _hash_corners(gx, gy, gz)
                ws = _corner_w(fx, fy, fz)
                acc0 = jnp.zeros((16,), jnp.float32)
                acc1 = jnp.zeros((16,), jnp.float32)
                for k in range(8):
                    h2 = _mod_level(hs[k], _SIZES[l], _OFF[l])
                    v0 = plsc.load_gather(tab, [h2])
                    v1 = plsc.load_gather(tab, [h2 + one_i])
                    acc0 = acc0 + ws[k] * v0
                    acc1 = acc1 + ws[k] * v1
                plsc.store_scatter(ob, [pb + (3 + 2 * l)], acc0)
                plsc.store_scatter(ob, [pb + (4 + 2 * l)], acc1)
            return c2

        lax.fori_loop(0, _G, fused, 0)

        def make_p1(l):
            res = _RES[l]
            off = _OFF[l]
            size = _SIZES[l]
            fxb, fyb, fzb = fbufs[l % 2]
            idxr = ibufs[l % 2]

            def p1(g, c2):
                s16 = pl.ds(g * 16, 16)
                gx, fx = _prep(xb[s16], res)
                gy, fy = _prep(yb[s16], res)
                gz, fz = _prep(zb[s16], res)
                fxb[s16] = fx
                fyb[s16] = fy
                fzb[s16] = fz
                hs = _hash_corners(gx, gy, gz)
                for k in range(8):
                    h2 = _mod_level(hs[k], size, off)
                    idxr[pl.ds(2 * k * _C + g * 16, 16)] = h2
                    idxr[pl.ds((2 * k + 1) * _C + g * 16, 16)] = h2 + one_i
                return c2

            return p1

        def make_p2(l):
            fxb, fyb, fzb = fbufs[l % 2]
            vals = vbufs[l % 2]

            def p2(g, c2):
                s16 = pl.ds(g * 16, 16)
                ws = _corner_w(fxb[s16], fyb[s16], fzb[s16])
                acc0 = jnp.zeros((16,), jnp.float32)
                acc1 = jnp.zeros((16,), jnp.float32)
                r0 = g * 16 + iota
                for k in range(8):
                    v0 = plsc.load_gather(vals, [r0 + 2 * k * _C])
                    v1 = plsc.load_gather(vals, [r0 + (2 * k + 1) * _C])
                    acc0 = acc0 + ws[k] * v0
                    acc1 = acc1 + ws[k] * v1
                pb = iota35 + g * (16 * _OUT_D)
                plsc.store_scatter(ob, [pb + (3 + 2 * l)], acc0)
                plsc.store_scatter(ob, [pb + (4 + 2 * l)], acc1)
                return c2

            return p2

        def start_gather(l):
            return pltpu.async_copy(emb.at[ibufs[l % 2]], vbufs[l % 2],
                                    sems[l % 2])

        l0 = _N_RES_LEVELS
        lax.fori_loop(0, _G, make_p1(l0), 0)
        handle = start_gather(l0)
        for l in range(l0, _N_LEVELS):
            nxt = None
            if l + 1 < _N_LEVELS:
                lax.fori_loop(0, _G, make_p1(l + 1), 0)
                nxt = start_gather(l + 1)
            handle.wait()
            lax.fori_loop(0, _G, make_p2(l), 0)
            handle = nxt

        pltpu.sync_copy(ob, out.at[pl.ds(base * _OUT_D, _C * _OUT_D)])
        return carry

    lax.fori_loop(0, _NCH, chunk_body, 0)


_sc_call = pl.kernel(
    _body,
    out_type=jax.ShapeDtypeStruct((_B * _OUT_D,), jnp.float32),
    mesh=plsc.VectorSubcoreMesh(core_axis_name="c", subcore_axis_name="s"),
    compiler_params=pltpu.CompilerParams(
        needs_layout_passes=False, use_tc_tiling_on_sc=False),
    scratch_types=[
        pltpu.VMEM((_C,), jnp.float32),
        pltpu.VMEM((_C,), jnp.float32),
        pltpu.VMEM((_C,), jnp.float32),
        pltpu.VMEM((_C,), jnp.float32),
        pltpu.VMEM((_C,), jnp.float32),
        pltpu.VMEM((_C,), jnp.float32),
        pltpu.VMEM((_C,), jnp.float32),
        pltpu.VMEM((_C,), jnp.float32),
        pltpu.VMEM((_C,), jnp.float32),
        pltpu.VMEM((2 * 8 * _C,), jnp.int32),
        pltpu.VMEM((2 * 8 * _C,), jnp.int32),
        pltpu.VMEM((2 * 8 * _C,), jnp.float32),
        pltpu.VMEM((2 * 8 * _C,), jnp.float32),
        pltpu.VMEM((_TAB_WORDS,), jnp.float32),
        pltpu.VMEM((_C * _OUT_D,), jnp.float32),
        pltpu.SemaphoreType.DMA,
        pltpu.SemaphoreType.DMA,
    ],
)


@jax.jit
def kernel(xyz, embeddings):
    x = xyz[:, 0]
    y = xyz[:, 1]
    z = xyz[:, 2]
    emb_flat = embeddings.reshape(-1)
    flat = _sc_call(x, y, z, emb_flat)
    return flat.reshape(_B, _OUT_D)
